# initial kernel scaffold (unmeasured)
import jax
import jax.numpy as jnp
from jax import lax
from jax.experimental import pallas as pl
from jax.experimental.pallas import tpu as pltpu

N_DEV = 16
M = 512
CH = M // N_DEV
N_STEPS = N_DEV - 1


def kernel(dy, W):
    m, k = dy.shape

    def body(dy_ref, w_ref, out_ref, recv_buf, send_sems, recv_sems):
        my = lax.axis_index("i")
        right = lax.rem(my + 1, N_DEV)

        out_ref[...] = lax.dot_general(
            dy_ref[...],
            w_ref[...],
            dimension_numbers=(((1,), (1,)), ((), ())),
            preferred_element_type=jnp.float32,
        )

        for s in range(N_STEPS):
            c_send = lax.rem(my - s + 2 * N_DEV, N_DEV)
            c_recv = lax.rem(my - s - 1 + 2 * N_DEV, N_DEV)
            rdma = pltpu.make_async_remote_copy(
                src_ref=out_ref.at[pl.ds(c_send * CH, CH), :],
                dst_ref=recv_buf.at[s],
                send_sem=send_sems.at[s],
                recv_sem=recv_sems.at[s],
                device_id=(right,),
                device_id_type=pl.DeviceIdType.MESH,
            )
            rdma.start()
            rdma.wait()
            out_ref[pl.ds(c_recv * CH, CH), :] = (
                out_ref[pl.ds(c_recv * CH, CH), :] + recv_buf[s]
            )

        for s in range(N_STEPS):
            c_send = lax.rem(my + 1 - s + 2 * N_DEV, N_DEV)
            slot = N_STEPS + s
            rdma = pltpu.make_async_remote_copy(
                src_ref=out_ref.at[pl.ds(c_send * CH, CH), :],
                dst_ref=out_ref.at[pl.ds(c_send * CH, CH), :],
                send_sem=send_sems.at[slot],
                recv_sem=recv_sems.at[slot],
                device_id=(right,),
                device_id_type=pl.DeviceIdType.MESH,
            )
            rdma.start()
            rdma.wait()

    out_shape = jax.ShapeDtypeStruct((M, M), jnp.float32)
    return pl.pallas_call(
        body,
        out_shape=out_shape,
        in_specs=[
            pl.BlockSpec(memory_space=pltpu.VMEM),
            pl.BlockSpec(memory_space=pltpu.VMEM),
        ],
        out_specs=pl.BlockSpec(memory_space=pltpu.VMEM),
        scratch_shapes=[
            pltpu.VMEM((N_STEPS, CH, M), jnp.float32),
            pltpu.SemaphoreType.DMA((2 * N_STEPS,)),
            pltpu.SemaphoreType.DMA((2 * N_STEPS,)),
        ],
        compiler_params=pltpu.CompilerParams(collective_id=0),
    )(dy, W)


# baseline (device time: 92659 ns/iter reference)
import jax
import jax.numpy as jnp
from jax import lax
from jax.experimental import pallas as pl
from jax.experimental.pallas import tpu as pltpu

N_DEV = 16
M = 512
CH = M // N_DEV
N_STEPS = N_DEV - 1


def kernel(dy, W):
    m, k = dy.shape

    def body(dy_ref, w_ref, out_ref, recv_buf, send_sems, recv_sems):
        my = lax.axis_index("i")
        right = lax.rem(my + 1, N_DEV)

        out_ref[...] = lax.dot_general(
            dy_ref[...],
            w_ref[...],
            dimension_numbers=(((1,), (1,)), ((), ())),
            preferred_element_type=jnp.float32,
        )

        for s in range(N_STEPS):
            c_send = lax.rem(my - s + 2 * N_DEV, N_DEV)
            c_recv = lax.rem(my - s - 1 + 2 * N_DEV, N_DEV)
            rdma = pltpu.make_async_remote_copy(
                src_ref=out_ref.at[pl.ds(c_send * CH, CH), :],
                dst_ref=recv_buf.at[s],
                send_sem=send_sems.at[s],
                recv_sem=recv_sems.at[s],
                device_id=(right,),
                device_id_type=pl.DeviceIdType.MESH,
            )
            rdma.start()
            rdma.wait()
            out_ref[pl.ds(c_recv * CH, CH), :] = (
                out_ref[pl.ds(c_recv * CH, CH), :] + recv_buf[s]
            )

        for s in range(N_STEPS):
            c_send = lax.rem(my + 1 - s + 2 * N_DEV, N_DEV)
            slot = N_STEPS + s
            rdma = pltpu.make_async_remote_copy(
                src_ref=out_ref.at[pl.ds(c_send * CH, CH), :],
                dst_ref=out_ref.at[pl.ds(c_send * CH, CH), :],
                send_sem=send_sems.at[slot],
                recv_sem=recv_sems.at[slot],
                device_id=(right,),
                device_id_type=pl.DeviceIdType.MESH,
            )
            rdma.start()
            rdma.wait()

    out_shape = jax.ShapeDtypeStruct((M, M), jnp.float32)
    return pl.pallas_call(
        body,
        out_shape=out_shape,
        in_specs=[
            pl.BlockSpec(memory_space=pltpu.VMEM),
            pl.BlockSpec(memory_space=pltpu.VMEM),
        ],
        out_specs=pl.BlockSpec(memory_space=pltpu.VMEM),
        scratch_shapes=[
            pltpu.VMEM((N_STEPS, CH, M), jnp.float32),
            pltpu.SemaphoreType.DMA((2 * N_STEPS,)),
            pltpu.SemaphoreType.DMA((2 * N_STEPS,)),
        ],
    )(dy, W)


# device time: 38510 ns/iter; 2.4061x vs baseline; 2.4061x over previous
import jax
import jax.numpy as jnp
from jax import lax
from jax.experimental import pallas as pl
from jax.experimental.pallas import tpu as pltpu

N_DEV = 16
M = 512
CH = M // N_DEV


def kernel(dy, W):
    def body(dy_ref, w_ref, out_ref, acc_ref, rs_recv,
             rs_send_sems, rs_recv_sems, ag_send_sems, ag_recv_sems):
        my = lax.axis_index("i")

        acc_ref[...] = lax.dot_general(
            dy_ref[...],
            w_ref[...],
            dimension_numbers=(((1,), (1,)), ((), ())),
            preferred_element_type=jnp.float32,
        )

        rs_rdmas = []
        for d in range(1, N_DEV):
            p = lax.rem(my + d, N_DEV)
            rdma = pltpu.make_async_remote_copy(
                src_ref=acc_ref.at[pl.ds(p * CH, CH), :],
                dst_ref=rs_recv.at[my],
                send_sem=rs_send_sems.at[p],
                recv_sem=rs_recv_sems.at[my],
                device_id=(p,),
                device_id_type=pl.DeviceIdType.MESH,
            )
            rdma.start()
            rs_rdmas.append(rdma)

        rs_recv[my] = acc_ref[pl.ds(my * CH, CH), :]
        for d in range(1, N_DEV):
            p = lax.rem(my + d, N_DEV)
            recv = pltpu.make_async_remote_copy(
                src_ref=rs_recv.at[p],
                dst_ref=rs_recv.at[p],
                send_sem=rs_recv_sems.at[p],
                recv_sem=rs_recv_sems.at[p],
                device_id=(my,),
                device_id_type=pl.DeviceIdType.MESH,
            )
            recv.wait_recv()
        out_ref[pl.ds(my * CH, CH), :] = jnp.sum(rs_recv[...], axis=0)

        ag_rdmas = []
        for d in range(1, N_DEV):
            p = lax.rem(my + d, N_DEV)
            rdma = pltpu.make_async_remote_copy(
                src_ref=out_ref.at[pl.ds(my * CH, CH), :],
                dst_ref=out_ref.at[pl.ds(my * CH, CH), :],
                send_sem=ag_send_sems.at[p],
                recv_sem=ag_recv_sems.at[my],
                device_id=(p,),
                device_id_type=pl.DeviceIdType.MESH,
            )
            rdma.start()
            ag_rdmas.append(rdma)

        for d in range(1, N_DEV):
            p = lax.rem(my + d, N_DEV)
            recv = pltpu.make_async_remote_copy(
                src_ref=rs_recv.at[0],
                dst_ref=out_ref.at[pl.ds(p * CH, CH), :],
                send_sem=ag_recv_sems.at[p],
                recv_sem=ag_recv_sems.at[p],
                device_id=(my,),
                device_id_type=pl.DeviceIdType.MESH,
            )
            recv.wait_recv()

        for rdma in rs_rdmas:
            rdma.wait_send()
        for rdma in ag_rdmas:
            rdma.wait_send()

    out_shape = jax.ShapeDtypeStruct((M, M), jnp.float32)
    return pl.pallas_call(
        body,
        out_shape=out_shape,
        in_specs=[
            pl.BlockSpec(memory_space=pltpu.VMEM),
            pl.BlockSpec(memory_space=pltpu.VMEM),
        ],
        out_specs=pl.BlockSpec(memory_space=pltpu.VMEM),
        scratch_shapes=[
            pltpu.VMEM((M, M), jnp.float32),
            pltpu.VMEM((N_DEV, CH, M), jnp.float32),
            pltpu.SemaphoreType.DMA((N_DEV,)),
            pltpu.SemaphoreType.DMA((N_DEV,)),
            pltpu.SemaphoreType.DMA((N_DEV,)),
            pltpu.SemaphoreType.DMA((N_DEV,)),
        ],
    )(dy, W)
